# baseline (speedup 1.0000x reference)
import functools

import jax
import jax.numpy as jnp
from jax import lax
from jax.experimental import pallas as pl
from jax.experimental.pallas import tpu as pltpu
from jax.experimental.pallas import tpu_sc as plsc

_VOCAB = 100000
_EMBD = 64
_BATCH = 4096
_MAX_TIME = 200

_NW = 32
_BLK = _BATCH // _NW


def _transpose_128_to(src, dst, e_rows):
    @pl.loop(0, e_rows)
    def _row(e):
        eidx = jnp.zeros((16,), jnp.int32) + e
        for g in range(8):
            bidx = jnp.arange(16, dtype=jnp.int32) + (g * 16)
            dst[e, pl.ds(g * 16, 16)] = plsc.load_gather(src, [bidx, eidx])


def _body(inputs_t_hbm, lens_hbm, table_hbm, out_hbm, fs_hbm,
          idx_v, rows_v, tr_v, lens_v, ids_v,
          gsem0, gsem1, ssem0, ssem1, fsem):
    wid = lax.axis_index("s") * 2 + lax.axis_index("c")
    b0 = wid * _BLK
    gsems = (gsem0, gsem1)
    ssems = (ssem0, ssem1)

    pltpu.sync_copy(inputs_t_hbm.at[:, pl.ds(b0, _BLK)], idx_v)

    def start_gather(t, b):
        pltpu.async_copy(table_hbm.at[idx_v.at[t]], rows_v.at[b], gsems[b])

    def wait_gather(b):
        pltpu.make_async_copy(
            table_hbm.at[idx_v.at[0]], rows_v.at[b], gsems[b]).wait()

    def start_scatter(t, b):
        pltpu.async_copy(
            tr_v.at[b], out_hbm.at[t, :, pl.ds(b0, _BLK)], ssems[b])

    def wait_scatter(b):
        pltpu.make_async_copy(
            tr_v.at[b], out_hbm.at[0, :, pl.ds(b0, _BLK)], ssems[b]).wait()

    start_gather(0, 0)

    @pl.loop(0, _MAX_TIME // 2)
    def _t_pair(i):
        for b in range(2):
            t = i * 2 + b
            wait_gather(b)

            @pl.when(t + 1 < _MAX_TIME)
            def _():
                start_gather(t + 1, 1 - b)

            @pl.when(t >= 2)
            def _():
                wait_scatter(b)

            _transpose_128_to(rows_v.at[b], tr_v.at[b], _EMBD)
            start_scatter(t, b)

    for b in range(2):
        wait_scatter(b)

    pltpu.sync_copy(lens_hbm.at[pl.ds(b0, _BLK)], lens_v)
    for i in range(_BLK // 16):
        lens = lens_v[pl.ds(i * 16, 16)]
        last = jnp.clip(lens - 1, 0, _MAX_TIME - 1)
        col = jnp.arange(16, dtype=jnp.int32) + (i * 16)
        ids_v[pl.ds(i * 16, 16)] = plsc.load_gather(idx_v, [last, col])
    pltpu.async_copy(table_hbm.at[ids_v], rows_v.at[0], fsem).wait()
    _transpose_128_to(rows_v.at[0], tr_v.at[0], _EMBD)
    pltpu.sync_copy(tr_v.at[0], fs_hbm.at[:, pl.ds(b0, _BLK)])


@functools.cache
def _build():
    mesh = plsc.VectorSubcoreMesh(core_axis_name="c", subcore_axis_name="s")
    return pl.kernel(
        _body,
        out_type=(
            jax.ShapeDtypeStruct((_MAX_TIME, _EMBD, _BATCH), jnp.float32),
            jax.ShapeDtypeStruct((_EMBD, _BATCH), jnp.float32),
        ),
        mesh=mesh,
        scratch_types=[
            pltpu.VMEM((_MAX_TIME, _BLK), jnp.int32),
            pltpu.VMEM((2, _BLK, 128), jnp.float32),
            pltpu.VMEM((2, _EMBD, _BLK), jnp.float32),
            pltpu.VMEM((_BLK,), jnp.int32),
            pltpu.VMEM((_BLK,), jnp.int32),
            pltpu.SemaphoreType.DMA,
            pltpu.SemaphoreType.DMA,
            pltpu.SemaphoreType.DMA,
            pltpu.SemaphoreType.DMA,
            pltpu.SemaphoreType.DMA,
        ],
        compiler_params=pltpu.CompilerParams(
            use_tc_tiling_on_sc=True, needs_layout_passes=False),
    )


def kernel(inputs, input_lengths, table):
    inputs_t = inputs.T
    table_p = jnp.concatenate(
        [table, jnp.zeros_like(table)], axis=1)
    out_t, fs_t = _build()(inputs_t, input_lengths, table_p)
    return out_t.transpose(2, 0, 1), fs_t.T


# speedup vs baseline: 1.8102x; 1.8102x over previous
import functools

import jax
import jax.numpy as jnp
from jax import lax
from jax.experimental import pallas as pl
from jax.experimental.pallas import tpu as pltpu
from jax.experimental.pallas import tpu_sc as plsc

_VOCAB = 100000
_EMBD = 64
_BATCH = 4096
_MAX_TIME = 200

_NW = 32
_BLK = _BATCH // _NW


def _transpose_128_to(src, dst, e_rows):
    @plsc.parallel_loop(0, e_rows, unroll=4)
    def _row(e):
        eidx = jnp.zeros((16,), jnp.int32) + e
        for g in range(8):
            bidx = jnp.arange(16, dtype=jnp.int32) + (g * 16)
            dst[e, pl.ds(g * 16, 16)] = plsc.load_gather(src, [bidx, eidx])


def _body(inputs_t_hbm, lens_hbm, table_hbm, out_hbm, fs_hbm,
          idx_v, rows_v, tr_v, lens_v, ids_v,
          gsem0, gsem1, ssem0, ssem1, fsem):
    wid = lax.axis_index("s") * 2 + lax.axis_index("c")
    b0 = wid * _BLK
    gsems = (gsem0, gsem1)
    ssems = (ssem0, ssem1)

    pltpu.sync_copy(inputs_t_hbm.at[:, pl.ds(b0, _BLK)], idx_v)

    def start_gather(t, b):
        pltpu.async_copy(table_hbm.at[idx_v.at[t]], rows_v.at[b], gsems[b])

    def wait_gather(b):
        pltpu.make_async_copy(
            table_hbm.at[idx_v.at[0]], rows_v.at[b], gsems[b]).wait()

    def start_scatter(t, b):
        pltpu.async_copy(
            tr_v.at[b], out_hbm.at[t, :, pl.ds(b0, _BLK)], ssems[b])

    def wait_scatter(b):
        pltpu.make_async_copy(
            tr_v.at[b], out_hbm.at[0, :, pl.ds(b0, _BLK)], ssems[b]).wait()

    start_gather(0, 0)

    @pl.loop(0, _MAX_TIME // 2)
    def _t_pair(i):
        for b in range(2):
            t = i * 2 + b
            wait_gather(b)

            @pl.when(t + 1 < _MAX_TIME)
            def _():
                start_gather(t + 1, 1 - b)

            @pl.when(t >= 2)
            def _():
                wait_scatter(b)

            _transpose_128_to(rows_v.at[b], tr_v.at[b], _EMBD)
            start_scatter(t, b)

    for b in range(2):
        wait_scatter(b)

    pltpu.sync_copy(lens_hbm.at[pl.ds(b0, _BLK)], lens_v)
    for i in range(_BLK // 16):
        lens = lens_v[pl.ds(i * 16, 16)]
        last = jnp.clip(lens - 1, 0, _MAX_TIME - 1)
        col = jnp.arange(16, dtype=jnp.int32) + (i * 16)
        ids_v[pl.ds(i * 16, 16)] = plsc.load_gather(idx_v, [last, col])
    pltpu.async_copy(table_hbm.at[ids_v], rows_v.at[0], fsem).wait()
    _transpose_128_to(rows_v.at[0], tr_v.at[0], _EMBD)
    pltpu.sync_copy(tr_v.at[0], fs_hbm.at[:, pl.ds(b0, _BLK)])


@functools.cache
def _build():
    mesh = plsc.VectorSubcoreMesh(core_axis_name="c", subcore_axis_name="s")
    return pl.kernel(
        _body,
        out_type=(
            jax.ShapeDtypeStruct((_MAX_TIME, _EMBD, _BATCH), jnp.float32),
            jax.ShapeDtypeStruct((_EMBD, _BATCH), jnp.float32),
        ),
        mesh=mesh,
        scratch_types=[
            pltpu.VMEM((_MAX_TIME, _BLK), jnp.int32),
            pltpu.VMEM((2, _BLK, 128), jnp.float32),
            pltpu.VMEM((2, _EMBD, _BLK), jnp.float32),
            pltpu.VMEM((_BLK,), jnp.int32),
            pltpu.VMEM((_BLK,), jnp.int32),
            pltpu.SemaphoreType.DMA,
            pltpu.SemaphoreType.DMA,
            pltpu.SemaphoreType.DMA,
            pltpu.SemaphoreType.DMA,
            pltpu.SemaphoreType.DMA,
        ],
        compiler_params=pltpu.CompilerParams(
            use_tc_tiling_on_sc=True, needs_layout_passes=False),
    )


def kernel(inputs, input_lengths, table):
    inputs_t = inputs.T
    table_p = jnp.concatenate(
        [table, jnp.zeros_like(table)], axis=1)
    out_t, fs_t = _build()(inputs_t, input_lengths, table_p)
    return out_t.transpose(2, 0, 1), fs_t.T
